# baseline (speedup 1.0000x reference)
import jax
import jax.numpy as jnp
from jax import lax
from jax.experimental import pallas as pl
from jax.experimental.pallas import tpu as pltpu
from jax.experimental.pallas import tpu_sc as plsc

_ROWS = 16384
_COLS = 200
_NW = 32
_RPW = _ROWS // _NW
_CHUNK = 256


def _sc_body(x_hbm, o0, o1, o2, o3, o4, g, bufa, bufb, sem):
    c = lax.axis_index("c")
    s = lax.axis_index("s")
    wid = s * 2 + c
    base = wid * _RPW

    pending = []

    pltpu.sync_copy(x_hbm.at[pl.ds(base, _RPW), 0:8], g)

    zeros = jnp.zeros((16,), jnp.int32)
    ones = jnp.ones((16,), jnp.int32)

    def fix(i, carry):
        rows16 = i * 16 + lax.iota(jnp.int32, 16)
        c0 = plsc.load_gather(g, [rows16, zeros])
        c1 = plsc.load_gather(g, [rows16, ones])
        plsc.store_scatter(g, [rows16, zeros], c1)
        plsc.store_scatter(g, [rows16, ones], c0)
        return carry

    lax.fori_loop(0, _RPW // 16, fix, 0)

    for dst in (o1, o3):
        cp = pltpu.make_async_copy(g, dst.at[pl.ds(base, _RPW), 0:8], sem)
        cp.start()
        pending.append(cp)

    bufs = (bufa, bufb)
    for k in range(_RPW // _CHUNK):
        buf = bufs[k % 2]
        rows = pl.ds(base + k * _CHUNK, _CHUNK)
        pltpu.sync_copy(x_hbm.at[rows, :], buf)
        for dst in (o0, o2, o4):
            cp = pltpu.make_async_copy(buf, dst.at[rows, :], sem)
            cp.start()
            pending.append(cp)
        for dst in (o1, o3):
            cp = pltpu.make_async_copy(
                buf.at[:, 8:_COLS], dst.at[rows, 8:_COLS], sem
            )
            cp.start()
            pending.append(cp)

    for cp in pending:
        cp.wait()


def _make_sc_kernel():
    mesh = plsc.VectorSubcoreMesh(core_axis_name="c", subcore_axis_name="s")
    struct = jax.ShapeDtypeStruct((_ROWS, _COLS), jnp.float32)
    return pl.kernel(
        _sc_body,
        out_type=[struct] * 5,
        mesh=mesh,
        compiler_params=pltpu.CompilerParams(
            use_tc_tiling_on_sc=False, needs_layout_passes=False
        ),
        scratch_types=[
            pltpu.VMEM((_RPW, 8), jnp.float32),
            pltpu.VMEM((_CHUNK, _COLS), jnp.float32),
            pltpu.VMEM((_CHUNK, _COLS), jnp.float32),
            pltpu.SemaphoreType.DMA,
        ],
    )


_sc_perturb = _make_sc_kernel()


def kernel(x):
    o0, o1, o2, o3, o4 = _sc_perturb(x)
    return (o0, o1, o2, o3, o4)
